# baseline (speedup 1.0000x reference)
import functools

import jax
import jax.numpy as jnp
import numpy as np
from jax import lax
from jax.experimental import pallas as pl
from jax.experimental.pallas import tpu as pltpu
from jax.experimental.pallas import tpu_sc as plsc

K_PCT = 10
EPS_DICE = 1e-05
EPS_LOG = 0.0001

N = 2 * 1 * 128 * 128 * 128
NW = 32
PER_TILE = N // NW
CHUNK = 8192
NCHUNK = PER_TILE // CHUNK
UNROLL = 8
L = 16
BINS1 = 2048
BINS2 = 1024
COLS = 512
ROWS_PER_CHUNK = CHUNK // COLS
SIGN = np.int32(-2147483648)
MASK31 = np.int32(0x7FFFFFFF)

_mesh = plsc.VectorSubcoreMesh(core_axis_name="c", subcore_axis_name="s")


def _wid():
    return lax.axis_index("s") * 2 + lax.axis_index("c")


def _zero_hist(hist_v, nbins):
    zero_v = jnp.broadcast_to(jnp.int32(0), (L,))
    shift = (nbins // L).bit_length() - 1

    @plsc.parallel_loop(0, L * (nbins // L), 1, unroll=UNROLL)
    def _(i):
        hist_v[i >> shift, pl.ds((i & (nbins // L - 1)) * L, L)] = zero_v


def _pass1_body(x_hbm, t_hbm, keys_hbm, hist_hbm,
                xbuf0, xbuf1, tbuf0, tbuf1, kbuf0, kbuf1, hist_v,
                sx0, sx1, st0, st1, sk0, sk1):
    wid = _wid()
    base = wid * PER_TILE
    lanes = lax.iota(jnp.int32, L)
    ones = jnp.broadcast_to(jnp.int32(1), (L,))
    xbufs, tbufs, kbufs = (xbuf0, xbuf1), (tbuf0, tbuf1), (kbuf0, kbuf1)
    sxs, sts, sks = (sx0, sx1), (st0, st1), (sk0, sk1)

    def load(ci, b):
        off = base + ci * CHUNK
        pltpu.async_copy(x_hbm.at[pl.ds(off, CHUNK)], xbufs[b], sxs[b])
        pltpu.async_copy(t_hbm.at[pl.ds(off, CHUNK)], tbufs[b], sts[b])

    load(0, 0)
    load(1, 1)
    _zero_hist(hist_v, BINS1)
    row_base = wid * (PER_TILE // COLS)

    def outer(g, c):
        for b in range(2):
            ci = g * 2 + b
            pltpu.make_async_copy(
                x_hbm.at[pl.ds(0, CHUNK)], xbufs[b], sxs[b]).wait()
            pltpu.make_async_copy(
                t_hbm.at[pl.ds(0, CHUNK)], tbufs[b], sts[b]).wait()

            @pl.when(g > 0)
            def _():
                pltpu.make_async_copy(
                    kbufs[b],
                    keys_hbm.at[pl.ds(0, ROWS_PER_CHUNK)], sks[b]).wait()

            xb, tb, kb = xbufs[b], tbufs[b], kbufs[b]

            @plsc.parallel_loop(0, CHUNK // L, 1, unroll=UNROLL)
            def _(vi):
                s = vi * L
                xv = xb[pl.ds(s, L)]
                tv = tb[pl.ds(s, L)]
                pos = tv >= 0.5
                p = jnp.where(pos, xv, 1.0 - xv)
                key = lax.bitcast_convert_type(p, jnp.int32)
                d1 = key >> 19
                plsc.addupdate_scatter(hist_v, [lanes, d1], ones)
                kb[vi >> 5, pl.ds((vi & 31) << 4, L)] = jnp.where(
                    pos, key | SIGN, key)

            row_off = row_base + ci * ROWS_PER_CHUNK
            pltpu.async_copy(
                kbufs[b], keys_hbm.at[pl.ds(row_off, ROWS_PER_CHUNK)], sks[b])

            @pl.when(ci + 2 < NCHUNK)
            def _():
                load(ci + 2, b)

        return c

    lax.fori_loop(0, NCHUNK // 2, outer, 0)
    pltpu.make_async_copy(
        kbufs[0], keys_hbm.at[pl.ds(0, ROWS_PER_CHUNK)], sks[0]).wait()
    pltpu.make_async_copy(
        kbufs[1], keys_hbm.at[pl.ds(0, ROWS_PER_CHUNK)], sks[1]).wait()
    pltpu.sync_copy(hist_v, hist_hbm.at[wid])


_pass1 = pl.kernel(
    _pass1_body,
    out_type=[
        jax.ShapeDtypeStruct((N // COLS, COLS), jnp.int32),
        jax.ShapeDtypeStruct((NW, L, BINS1), jnp.int32),
    ],
    mesh=_mesh,
    scratch_types=[
        pltpu.VMEM((CHUNK,), jnp.float32),
        pltpu.VMEM((CHUNK,), jnp.float32),
        pltpu.VMEM((CHUNK,), jnp.float32),
        pltpu.VMEM((CHUNK,), jnp.float32),
        pltpu.VMEM((ROWS_PER_CHUNK, COLS), jnp.int32),
        pltpu.VMEM((ROWS_PER_CHUNK, COLS), jnp.int32),
        pltpu.VMEM((L, BINS1), jnp.int32),
        pltpu.SemaphoreType.DMA,
        pltpu.SemaphoreType.DMA,
        pltpu.SemaphoreType.DMA,
        pltpu.SemaphoreType.DMA,
        pltpu.SemaphoreType.DMA,
        pltpu.SemaphoreType.DMA,
    ],
    compiler_params=pltpu.CompilerParams(needs_layout_passes=False),
)


def _pass2_body(keys_hbm, j1_hbm, hist_hbm, kbuf0, kbuf1, jbuf, hist_v,
                sk0, sk1):
    wid = _wid()
    base = wid * PER_TILE
    lanes = lax.iota(jnp.int32, L)
    ones = jnp.broadcast_to(jnp.int32(1), (L,))
    kbufs, sks = (kbuf0, kbuf1), (sk0, sk1)
    row_base = wid * (PER_TILE // COLS)

    def load(ci, b):
        row_off = row_base + ci * ROWS_PER_CHUNK
        pltpu.async_copy(
            keys_hbm.at[pl.ds(row_off, ROWS_PER_CHUNK)], kbufs[b], sks[b])

    load(0, 0)
    load(1, 1)
    _zero_hist(hist_v, BINS2)
    pltpu.sync_copy(j1_hbm.at[pl.ds(0, L)], jbuf)
    j1v = jbuf[...]

    def outer(g, c):
        for b in range(2):
            ci = g * 2 + b
            pltpu.make_async_copy(
                keys_hbm.at[pl.ds(0, ROWS_PER_CHUNK)], kbufs[b], sks[b]).wait()
            kb = kbufs[b]

            @plsc.parallel_loop(0, CHUNK // L, 1, unroll=UNROLL)
            def _(vi):
                kp = kb[vi >> 5, pl.ds((vi & 31) << 4, L)] & MASK31
                m = (kp >> 19) == j1v
                d2 = (kp >> 9) & jnp.int32(0x3FF)
                plsc.addupdate_scatter(hist_v, [lanes, d2], ones, mask=m)

            @pl.when(ci + 2 < NCHUNK)
            def _():
                load(ci + 2, b)

        return c

    lax.fori_loop(0, NCHUNK // 2, outer, 0)
    pltpu.sync_copy(hist_v, hist_hbm.at[wid])


_pass2 = pl.kernel(
    _pass2_body,
    out_type=[jax.ShapeDtypeStruct((NW, L, BINS2), jnp.int32)],
    mesh=_mesh,
    scratch_types=[
        pltpu.VMEM((ROWS_PER_CHUNK, COLS), jnp.int32),
        pltpu.VMEM((ROWS_PER_CHUNK, COLS), jnp.int32),
        pltpu.VMEM((L,), jnp.int32),
        pltpu.VMEM((L, BINS2), jnp.int32),
        pltpu.SemaphoreType.DMA,
        pltpu.SemaphoreType.DMA,
    ],
    compiler_params=pltpu.CompilerParams(needs_layout_passes=False),
)


_TC_ROWS = 256
_TC_GRID = N // (_TC_ROWS * COLS)


def _tc_body(pref_ref, keys_ref, out_ref):
    i = pl.program_id(0)

    @pl.when(i == 0)
    def _():
        out_ref[0] = 0.0
        out_ref[1] = 0.0
        out_ref[2] = 0.0
        out_ref[3] = 0.0

    kf = keys_ref[...]
    kp = kf & MASK31
    tpos = kf < 0
    p = lax.bitcast_convert_type(kp, jnp.float32)
    res = -jnp.log(p + EPS_LOG)
    sel = (kp >> 19) < pref_ref[0]
    out_ref[0] += jnp.sum(jnp.where(sel, res, 0.0))
    out_ref[1] += jnp.sum(jnp.where(tpos, p, 0.0))
    out_ref[2] += jnp.sum(jnp.where(tpos, p, 1.0 - p))
    out_ref[3] += jnp.sum(jnp.where(tpos, 1.0, 0.0))


def _tc_stage(keys2d, j1vec):
    return pl.pallas_call(
        _tc_body,
        grid=(_TC_GRID,),
        in_specs=[
            pl.BlockSpec(memory_space=pltpu.SMEM),
            pl.BlockSpec((_TC_ROWS, COLS), lambda i: (i, 0)),
        ],
        out_specs=pl.BlockSpec(memory_space=pltpu.SMEM),
        out_shape=jax.ShapeDtypeStruct((4,), jnp.float32),
    )(j1vec, keys2d)


KF = float(N * K_PCT // 100)


def _cum_rows(cnt, nrows):
    lane = lax.broadcasted_iota(jnp.int32, (nrows, 128), 1)
    pc = cnt
    for s in (1, 2, 4, 8, 16, 32, 64):
        pc = pc + jnp.where(lane >= s, pltpu.roll(pc, s, 1), 0)
    row = lax.broadcasted_iota(jnp.int32, (nrows, 128), 0)
    rowtot = jnp.broadcast_to(pc[:, 127:128], (nrows, 128))
    inc = rowtot
    s = 1
    while s < nrows:
        inc = inc + jnp.where(row >= s, pltpu.roll(inc, s, 0), 0)
        s *= 2
    return pc + inc - rowtot


KI = N * K_PCT // 100


def _sel1_body(hist_ref, j1s_ref, b1_ref, j1v_ref):
    h = hist_ref[...].reshape(NW * L, BINS1)
    cnt = jnp.sum(h, axis=0).reshape(BINS1 // 128, 128)
    cum = _cum_rows(cnt, BINS1 // 128)
    lt = cum < KI
    j1 = jnp.sum(lt.astype(jnp.int32))
    below1 = jnp.max(jnp.where(lt, cum, 0))
    j1s_ref[0] = j1
    b1_ref[0] = below1
    j1v_ref[...] = jnp.broadcast_to(j1, (128,))


def _sel1(hist1):
    return pl.pallas_call(
        _sel1_body,
        in_specs=[pl.BlockSpec((NW, L, BINS1), lambda: (0, 0, 0))],
        out_specs=[
            pl.BlockSpec(memory_space=pltpu.SMEM),
            pl.BlockSpec(memory_space=pltpu.SMEM),
            pl.BlockSpec((128,), lambda: (0,)),
        ],
        out_shape=[
            jax.ShapeDtypeStruct((1,), jnp.int32),
            jax.ShapeDtypeStruct((1,), jnp.int32),
            jax.ShapeDtypeStruct((128,), jnp.int32),
        ],
    )(hist1)


def _final_body(hist_ref, sums_ref, j1s_ref, b1_ref, out_ref):
    h = hist_ref[...].reshape(NW * L, BINS2)
    cnt = jnp.sum(h, axis=0).reshape(BINS2 // 128, 128)
    below1 = b1_ref[0]
    cum = _cum_rows(cnt, BINS2 // 128) + below1
    lt = cum < KI
    j2 = jnp.sum(lt.astype(jnp.int32))
    count_lt = jnp.max(jnp.where(lt, cum, below1))
    r = jnp.float32(KI - count_lt)

    bidx = (lax.broadcasted_iota(jnp.int32, (BINS2 // 128, 128), 0) * 128
            + lax.broadcasted_iota(jnp.int32, (BINS2 // 128, 128), 1))
    keys_rep = (j1s_ref[0] << 19) | (bidx << 9) | 256
    res_rep = -jnp.log(
        lax.bitcast_convert_type(keys_rep, jnp.float32) + EPS_LOG)
    in_bin = jnp.sum(jnp.where(bidx < j2, cnt.astype(jnp.float32) * res_rep, 0.0))
    res_j2 = jnp.sum(jnp.where(bidx == j2, res_rep, 0.0))

    ce = (sums_ref[0] + in_bin + r * res_j2) / KF
    dc = 1.0 - 2.0 * (sums_ref[1] + EPS_DICE) / (
        sums_ref[2] + sums_ref[3] + EPS_DICE)
    out_ref[0] = ce + dc
    out_ref[1] = ce
    out_ref[2] = dc


def _final(hist2, sums, j1s, b1):
    return pl.pallas_call(
        _final_body,
        in_specs=[
            pl.BlockSpec((NW, L, BINS2), lambda: (0, 0, 0)),
            pl.BlockSpec(memory_space=pltpu.SMEM),
            pl.BlockSpec(memory_space=pltpu.SMEM),
            pl.BlockSpec(memory_space=pltpu.SMEM),
        ],
        out_specs=pl.BlockSpec(memory_space=pltpu.SMEM),
        out_shape=jax.ShapeDtypeStruct((3,), jnp.float32),
    )(hist2, sums, j1s, b1)


def kernel(net_output, target):
    x = net_output.reshape(-1)
    t = target.reshape(-1)

    keys, hist1 = _pass1(x, t)
    j1s, b1, j1vec = _sel1(hist1)
    (hist2,) = _pass2(keys, j1vec)
    sums = _tc_stage(keys, j1s)
    out3 = _final(hist2, sums, j1s, b1)
    return (out3[0], out3[1], out3[2])
